# baseline (speedup 1.0000x reference)
import functools

import jax
import jax.numpy as jnp
from jax import lax
from jax.experimental import pallas as pl
from jax.experimental.pallas import tpu as pltpu
from jax.experimental.pallas import tpu_sc as plsc

K = 8
NB = 512
NLEV = 10
NW = 32
LANES = 16


NCH = 4
NB_CH = NB // NCH


def _pyramid_kernel(ctx_ref, t_ref, prev_ref):
    t = pl.program_id(1)
    neg = jnp.finfo(t_ref.dtype).min
    D = t_ref.shape[-1]

    @pl.when(t < NCH)
    def _bm():
        prev_ref[pl.ds(t * NB_CH, NB_CH), :] = jnp.max(ctx_ref[0], axis=1)

    @pl.when(t == NCH)
    def _lvl0():
        t_ref[0, 0] = prev_ref[...]

    for jj in range(1, NLEV):
        @pl.when(t == NCH + jj)
        def _lvl(sh=1 << (jj - 1)):
            prev = prev_ref[...]
            shifted = jnp.concatenate(
                [prev[sh:], jnp.full((sh, D), neg, prev.dtype)], axis=0)
            new = jnp.maximum(prev, shifted)
            t_ref[0, 0] = new
            prev_ref[...] = new


def _build_pyramid(context):
    B, S, D = context.shape
    ctx4 = context.reshape(B, NB, K, D)
    t = pl.pallas_call(
        _pyramid_kernel,
        grid=(B, NCH + NLEV),
        in_specs=[pl.BlockSpec(
            (1, NB_CH, K, D),
            lambda b, t: (b, jnp.minimum(t, NCH - 1), 0, 0))],
        out_specs=pl.BlockSpec(
            (1, 1, NB, D),
            lambda b, t: (b, jnp.clip(t - NCH, 0, NLEV - 1), 0, 0)),
        out_shape=jax.ShapeDtypeStruct((B, NLEV, NB, D), context.dtype),
        scratch_shapes=[pltpu.VMEM((NB, D), context.dtype)],
    )(ctx4)
    return t.reshape(B * NLEV * NB, D)


def _sc_span_kernel(nspans_w, S, D, ctx_hbm, t_hbm, s_hbm, e_hbm, out_hbm,
                    s_v, e_v, ctx_rows0, tbl_rows0, ctx_rows1, tbl_rows1,
                    tbl_idx0, tbl_idx1, pooled,
                    sem1, semc0, semt0, semc1, semt1):
    wid = lax.axis_index("s") * 2 + lax.axis_index("c")
    base_span = wid * nspans_w
    b = base_span // 256
    base_ctx = b * S
    base_t = b * (NLEV * NB)

    pltpu.async_copy(s_hbm.at[pl.ds(base_span, nspans_w)], s_v, sem1).wait()
    pltpu.async_copy(e_hbm.at[pl.ds(base_span, nspans_w)], e_v, sem1).wait()

    lane = lax.broadcasted_iota(jnp.int32, (LANES,), 0)
    nchunk = D // LANES

    def vfull(x):
        return jnp.full((LANES,), x, jnp.int32)

    bcast_dn = lax.GatherDimensionNumbers(
        offset_dims=(), collapsed_slice_dims=(0,), start_index_map=(0,))

    negf = jnp.full((LANES,), jnp.finfo(jnp.float32).min, jnp.float32)

    def compute_idx(sp):
        grp = pl.multiple_of((sp // LANES) * LANES, LANES)
        posv = vfull(sp - grp)
        s_chunk = s_v[pl.ds(grp, LANES)]
        e_chunk = e_v[pl.ds(grp, LANES)]
        sv = lax.gather(s_chunk, posv[:, None], bcast_dn, (1,),
                        mode=lax.GatherScatterMode.PROMISE_IN_BOUNDS)
        ev = lax.gather(e_chunk, posv[:, None], bcast_dn, (1,),
                        mode=lax.GatherScatterMode.PROMISE_IN_BOUNDS)

        one = vfull(1)
        zeroi = vfull(0)
        sbv = jnp.right_shift(sv + (K - 1), 3)
        ebv = jnp.right_shift(ev, 3)
        nbv = ebv - sbv
        hiv = jnp.where(nbv >= one, one, zeroi)

        jv = zeroi
        for kk in (2, 4, 8, 16, 32, 64, 128, 256, 512):
            jv = jv + jnp.where(nbv >= vfull(kk), one, zeroi)
        pjv = jnp.left_shift(one, jv)
        i1v = vfull(base_t) + (jv * NB + sbv) * hiv
        i2v = vfull(base_t) + (jv * NB + ebv - pjv) * hiv
        tidx = jnp.where(lane < one, i1v, i2v)

        n1v = jnp.minimum(ev, sbv * K) - sv
        r0v = jnp.maximum(sv, ebv * K)
        n2v = ev - r0v
        cidx = jnp.where(lane < n1v, sv + lane,
                         jnp.where(lane < n1v + n2v, r0v + lane - n1v, sv))
        cidx = cidx + vfull(base_ctx)

        int_f = hiv.astype(jnp.float32)
        nonemp_f = jnp.where(sv == ev, zeroi, one).astype(jnp.float32)
        return cidx, tidx, int_f, nonemp_f

    def reduce_span(sp, ctx_rows, tbl_rows, int_f, nonemp_f, nrows):
        def chunk_body(k, carry2):
            off = pl.multiple_of(k * LANES, LANES)
            m = [ctx_rows[r, pl.ds(off, LANES)] for r in range(nrows)]
            while len(m) > 1:
                m = [jnp.maximum(m[2 * i], m[2 * i + 1])
                     for i in range(len(m) // 2)]
            tacc = jnp.maximum(tbl_rows[0, pl.ds(off, LANES)],
                               tbl_rows[1, pl.ds(off, LANES)])
            tacc_eff = tacc * int_f + negf * (1.0 - int_f)
            val = jnp.maximum(m[0], tacc_eff) * nonemp_f
            pooled[sp, pl.ds(off, LANES)] = val
            return carry2

        lax.fori_loop(0, nchunk, chunk_body, 0, unroll=2)

    def pair_body(g, carry):
        sp0 = g * 2
        sp1 = sp0 + 1
        cidx0, tidx0, if0, nf0 = compute_idx(sp0)
        tbl_idx0[...] = tidx0
        c0 = pltpu.async_copy(ctx_hbm.at[cidx0], ctx_rows0, semc0)
        t0 = pltpu.async_copy(t_hbm.at[tbl_idx0.at[pl.ds(0, 2)]],
                              tbl_rows0, semt0)
        cidx1, tidx1, if1, nf1 = compute_idx(sp1)
        tbl_idx1[...] = tidx1
        c1 = pltpu.async_copy(ctx_hbm.at[cidx1], ctx_rows1, semc1)
        t1 = pltpu.async_copy(t_hbm.at[tbl_idx1.at[pl.ds(0, 2)]],
                              tbl_rows1, semt1)
        c0.wait()
        t0.wait()
        reduce_span(sp0, ctx_rows0, tbl_rows0, if0, nf0, LANES)
        c1.wait()
        t1.wait()
        reduce_span(sp1, ctx_rows1, tbl_rows1, if1, nf1, LANES)
        return carry

    lax.fori_loop(0, nspans_w // 2, pair_body, 0)
    pltpu.async_copy(pooled, out_hbm.at[pl.ds(base_span, nspans_w)],
                     sem1).wait()


def _sc_pool(ctxflat, tflat, s_arr, e_arr, S, D):
    n_spans = s_arr.shape[0]
    nspans_w = n_spans // NW
    mesh = plsc.VectorSubcoreMesh(core_axis_name="c", subcore_axis_name="s")
    sc_call = functools.partial(
        pl.kernel,
        out_type=jax.ShapeDtypeStruct((n_spans, D), ctxflat.dtype),
        mesh=mesh,
        scratch_types=[
            pltpu.VMEM((nspans_w,), jnp.int32),
            pltpu.VMEM((nspans_w,), jnp.int32),
            pltpu.VMEM((LANES, D), jnp.float32),
            pltpu.VMEM((2, D), jnp.float32),
            pltpu.VMEM((LANES, D), jnp.float32),
            pltpu.VMEM((2, D), jnp.float32),
            pltpu.VMEM((LANES,), jnp.int32),
            pltpu.VMEM((LANES,), jnp.int32),
            pltpu.VMEM((nspans_w, D), jnp.float32),
            pltpu.SemaphoreType.DMA,
            pltpu.SemaphoreType.DMA,
            pltpu.SemaphoreType.DMA,
            pltpu.SemaphoreType.DMA,
            pltpu.SemaphoreType.DMA,
        ],
    )(functools.partial(_sc_span_kernel, nspans_w, S, D))
    return sc_call(ctxflat, tflat, s_arr, e_arr)


def kernel(context, spans):
    B, S, D = context.shape
    NS = spans.shape[1]
    spans32 = spans.astype(jnp.int32)
    s_arr = spans32[:, :, 0].reshape(B * NS)
    e_arr = spans32[:, :, 1].reshape(B * NS)

    tflat = _build_pyramid(context)
    ctxflat = context.reshape(B * S, D)
    pooled = _sc_pool(ctxflat, tflat, s_arr, e_arr, S, D)

    res = jnp.zeros((B, S, D), dtype=context.dtype)
    return res.at[:, :NS, :].set(pooled.reshape(B, NS, D))
